# baseline (speedup 1.0000x reference)
import functools

import jax
import jax.numpy as jnp
from jax import lax
from jax.experimental import pallas as pl
from jax.experimental.pallas import tpu as pltpu
from jax.experimental.pallas import tpu_sc as plsc

N_TOTAL = 16777216
NC = 2
NS = 16
NW = NC * NS
PER_W = N_TOTAL // NW
CHUNK = 16384
NCHUNK = PER_W // CHUNK
LANES = 16


def _sc_partials(pred, target):
    mesh = plsc.VectorSubcoreMesh(core_axis_name="c", subcore_axis_name="s")

    @functools.partial(
        pl.kernel,
        mesh=mesh,
        out_type=jax.ShapeDtypeStruct((NW, 48), jnp.float32),
        scratch_types=[
            pltpu.VMEM((CHUNK,), jnp.float32),
            pltpu.VMEM((CHUNK,), jnp.float32),
            pltpu.VMEM((48,), jnp.float32),
            pltpu.SemaphoreType.DMA,
        ],
    )
    def k(pred_hbm, target_hbm, out_hbm, pbuf, tbuf, outv, sem):
        wid = lax.axis_index("s") * NC + lax.axis_index("c")
        base = wid * PER_W

        def chunk_body(j, carry):
            ape_acc, cnt_acc, sq_acc = carry
            off = base + j * CHUNK
            cp = pltpu.async_copy(pred_hbm.at[pl.ds(off, CHUNK)], pbuf, sem)
            ct = pltpu.async_copy(target_hbm.at[pl.ds(off, CHUNK)], tbuf, sem)
            cp.wait()
            ct.wait()

            def vec_body(i, c):
                a, cn, s = c
                t = tbuf[pl.ds(i * LANES, LANES)]
                p = pbuf[pl.ds(i * LANES, LANES)]
                diff = t - p
                s = s + diff * diff
                mask = t > 0.2
                a = a + jnp.where(mask, jnp.abs(diff) / t, 0.0)
                cn = cn + jnp.where(mask, 1.0, 0.0)
                return (a, cn, s)

            zero = jnp.zeros((LANES,), jnp.float32)
            a, cn, s = lax.fori_loop(
                0, CHUNK // LANES, vec_body, (zero, zero, zero))
            return (ape_acc + a, cnt_acc + cn, sq_acc + s)

        zero = jnp.zeros((LANES,), jnp.float32)
        ape_acc, cnt_acc, sq_acc = lax.fori_loop(
            0, NCHUNK, chunk_body, (zero, zero, zero))
        outv[pl.ds(0, LANES)] = ape_acc
        outv[pl.ds(16, LANES)] = cnt_acc
        outv[pl.ds(32, LANES)] = sq_acc
        pltpu.sync_copy(outv, out_hbm.at[wid])

    return k(pred, target)


def kernel(pred, target):
    parts = _sc_partials(pred, target)
    parts = parts.reshape(NW, 3, LANES)
    ape_sum = jnp.sum(parts[:, 0, :])
    cnt = jnp.sum(parts[:, 1, :])
    sq_sum = jnp.sum(parts[:, 2, :])
    mape = ape_sum / cnt
    rmse = jnp.sqrt(sq_sum / N_TOTAL)
    return (mape, rmse)


# speedup vs baseline: 1.4239x; 1.4239x over previous
import functools

import numpy as np
import jax
import jax.numpy as jnp
from jax import lax
from jax.experimental import pallas as pl
from jax.experimental.pallas import tpu as pltpu
from jax.experimental.pallas import tpu_sc as plsc

N_TOTAL = 16777216
NC = 2
NS = 16
NW = NC * NS
PER_W = N_TOTAL // NW
CHUNK = 16384
NCHUNK = PER_W // CHUNK
HALF = NCHUNK // 2
LANES = 16
UNROLL = 4

_RCP_MAGIC = np.int32(0x7EF311C3)


def _rcp(t):
    ti = lax.bitcast_convert_type(t, jnp.int32)
    r = lax.bitcast_convert_type(_RCP_MAGIC - ti, jnp.float32)
    r = r * (2.0 - t * r)
    r = r * (2.0 - t * r)
    return r


def _sc_partials(pred, target):
    mesh = plsc.VectorSubcoreMesh(core_axis_name="c", subcore_axis_name="s")

    @functools.partial(
        pl.kernel,
        mesh=mesh,
        out_type=jax.ShapeDtypeStruct((NW, 48), jnp.float32),
        scratch_types=[
            pltpu.VMEM((CHUNK,), jnp.float32),
            pltpu.VMEM((CHUNK,), jnp.float32),
            pltpu.VMEM((CHUNK,), jnp.float32),
            pltpu.VMEM((CHUNK,), jnp.float32),
            pltpu.VMEM((48,), jnp.float32),
            pltpu.SemaphoreType.DMA,
            pltpu.SemaphoreType.DMA,
            pltpu.SemaphoreType.DMA,
            pltpu.SemaphoreType.DMA,
        ],
    )
    def k(pred_hbm, target_hbm, out_hbm,
          pb0, pb1, tb0, tb1, outv, sp0, sp1, st0, st1):
        wid = lax.axis_index("s") * NC + lax.axis_index("c")
        base = wid * PER_W

        pltpu.async_copy(pred_hbm.at[pl.ds(base, CHUNK)], pb0, sp0)
        pltpu.async_copy(target_hbm.at[pl.ds(base, CHUNK)], tb0, st0)
        pltpu.async_copy(pred_hbm.at[pl.ds(base + CHUNK, CHUNK)], pb1, sp1)
        pltpu.async_copy(target_hbm.at[pl.ds(base + CHUNK, CHUNK)], tb1, st1)

        def compute_chunk(pb, tb, carry):
            def body(kk, c):
                o = kk * (UNROLL * LANES)
                out = []
                for u in range(UNROLL):
                    a, cn, s = c[u]
                    t = tb[pl.ds(o + u * LANES, LANES)]
                    p = pb[pl.ds(o + u * LANES, LANES)]
                    diff = t - p
                    s = s + diff * diff
                    mask = t > 0.2
                    ape = jnp.abs(diff) * _rcp(t)
                    a = a + jnp.where(mask, ape, 0.0)
                    cn = cn + jnp.where(mask, 1.0, 0.0)
                    out.append((a, cn, s))
                return tuple(out)
            return lax.fori_loop(0, CHUNK // (UNROLL * LANES), body, carry)

        def loop_body(i, carry):
            off0 = base + (2 * i) * CHUNK
            pltpu.make_async_copy(
                pred_hbm.at[pl.ds(off0, CHUNK)], pb0, sp0).wait()
            pltpu.make_async_copy(
                target_hbm.at[pl.ds(off0, CHUNK)], tb0, st0).wait()
            carry = compute_chunk(pb0, tb0, carry)

            @pl.when(i < HALF - 1)
            def _():
                pltpu.async_copy(
                    pred_hbm.at[pl.ds(off0 + 2 * CHUNK, CHUNK)], pb0, sp0)
                pltpu.async_copy(
                    target_hbm.at[pl.ds(off0 + 2 * CHUNK, CHUNK)], tb0, st0)

            off1 = off0 + CHUNK
            pltpu.make_async_copy(
                pred_hbm.at[pl.ds(off1, CHUNK)], pb1, sp1).wait()
            pltpu.make_async_copy(
                target_hbm.at[pl.ds(off1, CHUNK)], tb1, st1).wait()
            carry = compute_chunk(pb1, tb1, carry)

            @pl.when(i < HALF - 1)
            def _():
                pltpu.async_copy(
                    pred_hbm.at[pl.ds(off1 + 2 * CHUNK, CHUNK)], pb1, sp1)
                pltpu.async_copy(
                    target_hbm.at[pl.ds(off1 + 2 * CHUNK, CHUNK)], tb1, st1)

            return carry

        zero = jnp.zeros((LANES,), jnp.float32)
        init = tuple((zero, zero, zero) for _ in range(UNROLL))
        final = lax.fori_loop(0, HALF, loop_body, init)

        ape_acc = final[0][0] + final[1][0] + final[2][0] + final[3][0]
        cnt_acc = final[0][1] + final[1][1] + final[2][1] + final[3][1]
        sq_acc = final[0][2] + final[1][2] + final[2][2] + final[3][2]
        outv[pl.ds(0, LANES)] = ape_acc
        outv[pl.ds(16, LANES)] = cnt_acc
        outv[pl.ds(32, LANES)] = sq_acc
        pltpu.sync_copy(outv, out_hbm.at[wid])

    return k(pred, target)


def kernel(pred, target):
    parts = _sc_partials(pred, target)
    parts = parts.reshape(NW, 3, LANES)
    ape_sum = jnp.sum(parts[:, 0, :])
    cnt = jnp.sum(parts[:, 1, :])
    sq_sum = jnp.sum(parts[:, 2, :])
    mape = ape_sum / cnt
    rmse = jnp.sqrt(sq_sum / N_TOTAL)
    return (mape, rmse)
